# baseline (speedup 1.0000x reference)
import functools

import jax
import jax.numpy as jnp
from jax import lax
from jax.experimental import pallas as pl
from jax.experimental.pallas import tpu as pltpu
from jax.experimental.pallas import tpu_sc as plsc

_B = 16384 * 26
_D = 64
_NC = 2
_NS = 16
_NW = _NC * _NS
_PER_W = _B // _NW
_CHUNK = 512
_NCHUNK = _PER_W // _CHUNK

_mesh = plsc.VectorSubcoreMesh(core_axis_name="c", subcore_axis_name="s")


@functools.partial(
    pl.kernel,
    mesh=_mesh,
    out_type=jax.ShapeDtypeStruct((_B, _D), jnp.float32),
    scratch_types=[
        pltpu.VMEM((_CHUNK,), jnp.int32),
        pltpu.VMEM((_CHUNK, _D), jnp.float32),
        pltpu.SemaphoreType.DMA,
    ],
    compiler_params=pltpu.CompilerParams(use_tc_tiling_on_sc=False),
)
def _gather(idx_hbm, table_hbm, out_hbm, idx_v, rows_v, sem):
    wid = lax.axis_index("s") * _NC + lax.axis_index("c")
    base = wid * _PER_W

    def body(i, carry):
        off = base + i * _CHUNK
        pltpu.sync_copy(idx_hbm.at[pl.ds(off, _CHUNK)], idx_v)
        pltpu.async_copy(table_hbm.at[idx_v], rows_v, sem).wait()
        pltpu.sync_copy(rows_v, out_hbm.at[pl.ds(off, _CHUNK)])
        return carry

    lax.fori_loop(0, _NCHUNK, body, 0)


def kernel(indices, weight):
    flat = indices.reshape(-1).astype(jnp.int32)
    out = _gather(flat, weight)
    return out.reshape(indices.shape + (weight.shape[1],))


# speedup vs baseline: 1.0247x; 1.0247x over previous
import functools

import jax
import jax.numpy as jnp
from jax import lax
from jax.experimental import pallas as pl
from jax.experimental.pallas import tpu as pltpu
from jax.experimental.pallas import tpu_sc as plsc

_B = 16384 * 26
_D = 64
_NC = 2
_NS = 16
_NW = _NC * _NS
_PER_W = _B // _NW
_CHUNK = 832
_NCHUNK = _PER_W // _CHUNK

_mesh = plsc.VectorSubcoreMesh(core_axis_name="c", subcore_axis_name="s")


@functools.partial(
    pl.kernel,
    mesh=_mesh,
    out_type=jax.ShapeDtypeStruct((_B, _D), jnp.float32),
    scratch_types=[
        pltpu.VMEM((_NCHUNK, _CHUNK), jnp.int32),
        pltpu.VMEM((_CHUNK, _D), jnp.float32),
        pltpu.VMEM((_CHUNK, _D), jnp.float32),
        pltpu.SemaphoreType.DMA,
        pltpu.SemaphoreType.DMA,
        pltpu.SemaphoreType.DMA,
        pltpu.SemaphoreType.DMA,
    ],
    compiler_params=pltpu.CompilerParams(use_tc_tiling_on_sc=False),
)
def _gather(idx_hbm, table_hbm, out_hbm, idx_v, rows0, rows1, g0, g1, o0, o1):
    wid = lax.axis_index("s") * _NC + lax.axis_index("c")
    base = wid * _PER_W
    rows = (rows0, rows1)
    gsem = (g0, g1)
    osem = (o0, o1)

    pltpu.sync_copy(idx_hbm.at[wid], idx_v)

    def start_gather(c):
        b = c % 2
        pltpu.async_copy(table_hbm.at[idx_v.at[c]], rows[b], gsem[b])

    def wait_gather(c):
        b = c % 2
        pltpu.make_async_copy(table_hbm.at[idx_v.at[c]], rows[b], gsem[b]).wait()

    def start_out(c):
        b = c % 2
        pltpu.async_copy(rows[b], out_hbm.at[pl.ds(base + c * _CHUNK, _CHUNK)],
                         osem[b])

    def wait_out(c):
        b = c % 2
        pltpu.make_async_copy(
            rows[b], out_hbm.at[pl.ds(base + c * _CHUNK, _CHUNK)], osem[b]
        ).wait()

    start_gather(0)
    for c in range(_NCHUNK):
        wait_gather(c)
        start_out(c)
        if c >= 1 and c + 1 < _NCHUNK:
            wait_out(c - 1)
        if c + 1 < _NCHUNK:
            start_gather(c + 1)
    wait_out(_NCHUNK - 2)
    wait_out(_NCHUNK - 1)


def kernel(indices, weight):
    flat = indices.reshape(_NW, _NCHUNK, _CHUNK).astype(jnp.int32)
    out = _gather(flat, weight)
    return out.reshape(indices.shape + (weight.shape[1],))
